# baseline (speedup 1.0000x reference)
import jax
import jax.numpy as jnp
from jax.experimental import pallas as pl
from jax.experimental.pallas import tpu as pltpu

_EPS = 1e-9


def _rows_body(x_ref, u_ref, probs_ref, samp_ref):
    x = x_ref[...]
    u = u_ref[...]
    m = jnp.max(x, axis=-1, keepdims=True)
    e = jnp.exp(x - m)
    z = jnp.sum(e, axis=-1, keepdims=True)
    p = e / z
    probs_ref[...] = p
    g = -jnp.log(-jnp.log(u + _EPS) + _EPS)
    score = jnp.log(p + _EPS) + g
    samp_ref[...] = jnp.argmax(score, axis=-1, keepdims=True).astype(jnp.int32)


def kernel(logits, gumbel):
    B, T, V = logits.shape
    last = logits[:, T - 1, :]
    R = 8
    probs, samp = pl.pallas_call(
        _rows_body,
        grid=(B // R,),
        in_specs=[
            pl.BlockSpec((R, V), lambda b: (b, 0)),
            pl.BlockSpec((R, V), lambda b: (b, 0)),
        ],
        out_specs=[
            pl.BlockSpec((R, V), lambda b: (b, 0)),
            pl.BlockSpec((R, 1), lambda b: (b, 0)),
        ],
        out_shape=[
            jax.ShapeDtypeStruct((B, V), jnp.float32),
            jax.ShapeDtypeStruct((B, 1), jnp.int32),
        ],
    )(last, gumbel)
    return samp[:, 0], probs
